# baseline (speedup 1.0000x reference)
import functools

import jax
import jax.numpy as jnp
from jax import lax
from jax.experimental import pallas as pl
from jax.experimental.pallas import tpu as pltpu
from jax.experimental.pallas import tpu_sc as plsc


def _build_kernel(B, S, V, E):
    info = plsc.get_sparse_core_info()
    NC, NS, L = info.num_cores, info.num_subcores, info.num_lanes
    NW = NC * NS
    assert S % NW == 0
    s_per_w = S // NW
    CH = min(4, s_per_w)
    assert s_per_w % CH == 0 and E % L == 0
    NCH = s_per_w // CH
    NGEN = 4 if NCH % 4 == 0 else 1

    mesh = plsc.VectorSubcoreMesh(core_axis_name="c", subcore_axis_name="s")

    scratch = [pltpu.VMEM((B, s_per_w), jnp.int32)]
    scratch += [pltpu.VMEM((CH, E), jnp.float32) for _ in range(NGEN * B)]
    scratch += [pltpu.VMEM((CH, E), jnp.float32) for _ in range(NGEN)]
    scratch += [pltpu.SemaphoreType.DMA for _ in range(3 * NGEN + 1)]

    @functools.partial(
        pl.kernel,
        mesh=mesh,
        out_type=jax.ShapeDtypeStruct((B, S, E), jnp.float32),
        scratch_types=scratch,
    )
    def emb_kernel(x_hbm, tok_hbm, pos_hbm, out_hbm, idx_v, *rest):
        flat = list(rest[:NGEN * B])
        bufs = [flat[g * B:(g + 1) * B] for g in range(NGEN)]
        posb = list(rest[NGEN * B:NGEN * B + NGEN])
        sems = rest[NGEN * B + NGEN:]
        gsem = list(sems[:NGEN])
        osem = list(sems[NGEN:2 * NGEN])
        psem = list(sems[2 * NGEN:3 * NGEN])
        isem = sems[3 * NGEN]

        wid = lax.axis_index("s") * NC + lax.axis_index("c")
        s_base = wid * s_per_w

        def gather(c, g):
            return [
                pltpu.make_async_copy(
                    tok_hbm.at[idx_v.at[b, pl.ds(c * CH, CH)]],
                    bufs[g][b],
                    gsem[g],
                )
                for b in range(B)
            ]

        def pos(c, g):
            return pltpu.make_async_copy(
                pos_hbm.at[pl.ds(s_base + c * CH, CH)],
                posb[g],
                psem[g],
            )

        def out(c, g):
            return [
                pltpu.make_async_copy(
                    bufs[g][b],
                    out_hbm.at[b, pl.ds(s_base + c * CH, CH)],
                    osem[g],
                )
                for b in range(B)
            ]

        ihs = [
            pltpu.make_async_copy(
                x_hbm.at[b, pl.ds(s_base, s_per_w)], idx_v.at[b], isem
            )
            for b in range(B)
        ]
        for ih in ihs:
            ih.start()
        for ih in ihs:
            ih.wait()

        for g in range(2):
            pos(g, g).start()
            for d in gather(g, g):
                d.start()

        @pl.loop(0, NCH, step=NGEN)
        def step_group(c0):
            for g in range(NGEN):
                c = c0 + g
                for d in gather(c, g):
                    d.wait()
                pos(c, g).wait()

                bb, pv_ref = bufs[g], posb[g]

                @plsc.parallel_loop(0, CH)
                def add_row(r, bb=bb, pv_ref=pv_ref):
                    for j in range(E // L):
                        sl = pl.ds(j * L, L)
                        pv = pv_ref[r, sl]
                        for b in range(B):
                            plsc.addupdate(bb[b].at[r, sl], pv)

                for d in out(c, g):
                    d.start()

                @pl.when(c >= 2)
                def _(g=g):
                    for d in out(c - 2, (g - 2) % NGEN):
                        d.wait()

                @pl.when(c < NCH - 2)
                def _(g=g):
                    pos(c + 2, (g + 2) % NGEN).start()
                    for d in gather(c + 2, (g + 2) % NGEN):
                        d.start()

        for c in (NCH - 2, NCH - 1):
            for d in out(c, c % NGEN):
                d.wait()

    return emb_kernel


def kernel(x, tok_table, pos_table):
    B, S = x.shape
    V, E = tok_table.shape
    emb = _build_kernel(B, S, V, E)
    return emb(x.astype(jnp.int32), tok_table, pos_table)


# speedup vs baseline: 1.0918x; 1.0918x over previous
import functools

import jax
import jax.numpy as jnp
from jax import lax
from jax.experimental import pallas as pl
from jax.experimental.pallas import tpu as pltpu
from jax.experimental.pallas import tpu_sc as plsc


def _build_kernel(B, S, V, E):
    info = plsc.get_sparse_core_info()
    NC, NS, L = info.num_cores, info.num_subcores, info.num_lanes
    NW = NC * NS
    assert S % NW == 0
    s_per_w = S // NW
    CH = min(8, s_per_w)
    assert s_per_w % CH == 0 and E % L == 0
    NCH = s_per_w // CH
    NGEN = 2 if NCH % 2 == 0 else 1

    mesh = plsc.VectorSubcoreMesh(core_axis_name="c", subcore_axis_name="s")

    scratch = [pltpu.VMEM((B, s_per_w), jnp.int32)]
    scratch += [pltpu.VMEM((B * CH, E), jnp.float32) for _ in range(NGEN)]
    scratch += [pltpu.VMEM((CH, E), jnp.float32) for _ in range(NGEN)]
    scratch += [pltpu.SemaphoreType.DMA for _ in range(3 * NGEN + 1)]

    @functools.partial(
        pl.kernel,
        mesh=mesh,
        out_type=jax.ShapeDtypeStruct((B, S, E), jnp.float32),
        scratch_types=scratch,
    )
    def emb_kernel(x_hbm, tok_hbm, pos_hbm, out_hbm, idx_v, *rest):
        bufs = list(rest[:NGEN])
        posb = list(rest[NGEN:2 * NGEN])
        sems = rest[2 * NGEN:]
        gsem = list(sems[:NGEN])
        osem = list(sems[NGEN:2 * NGEN])
        psem = list(sems[2 * NGEN:3 * NGEN])
        isem = sems[3 * NGEN]

        wid = lax.axis_index("s") * NC + lax.axis_index("c")
        s_base = wid * s_per_w

        def gather(c, g):
            return [
                pltpu.make_async_copy(
                    tok_hbm.at[idx_v.at[b, pl.ds(c * CH, CH)]],
                    bufs[g].at[pl.ds(b * CH, CH)],
                    gsem[g],
                )
                for b in range(B)
            ]

        def pos(c, g):
            return pltpu.make_async_copy(
                pos_hbm.at[pl.ds(s_base + c * CH, CH)],
                posb[g],
                psem[g],
            )

        def out(c, g):
            return [
                pltpu.make_async_copy(
                    bufs[g].at[pl.ds(b * CH, CH)],
                    out_hbm.at[b, pl.ds(s_base + c * CH, CH)],
                    osem[g],
                )
                for b in range(B)
            ]

        ihs = [
            pltpu.make_async_copy(
                x_hbm.at[b, pl.ds(s_base, s_per_w)], idx_v.at[b], isem
            )
            for b in range(B)
        ]
        for ih in ihs:
            ih.start()
        for ih in ihs:
            ih.wait()

        for g in range(NGEN):
            pos(g, g).start()
            for d in gather(g, g):
                d.start()

        @pl.loop(0, NCH, step=NGEN)
        def step_group(c0):
            for g in range(NGEN):
                c = c0 + g
                for d in gather(c, g):
                    d.wait()
                pos(c, g).wait()

                buf, pv_ref = bufs[g], posb[g]

                @plsc.parallel_loop(0, CH)
                def add_row(r, buf=buf, pv_ref=pv_ref):
                    for j in range(E // L):
                        sl = pl.ds(j * L, L)
                        pv = pv_ref[r, sl]
                        for b in range(B):
                            plsc.addupdate(buf.at[b * CH + r, sl], pv)

                for d in out(c, g):
                    d.start()

                @pl.when(c < NCH - NGEN)
                def _():
                    pos(c + NGEN, g).start()

                for d in out(c, g):
                    d.wait()

                @pl.when(c < NCH - NGEN)
                def _():
                    for d in gather(c + NGEN, g):
                        d.start()

    return emb_kernel


def kernel(x, tok_table, pos_table):
    B, S = x.shape
    V, E = tok_table.shape
    emb = _build_kernel(B, S, V, E)
    return emb(x.astype(jnp.int32), tok_table, pos_table)


# speedup vs baseline: 1.1048x; 1.0119x over previous
import functools

import jax
import jax.numpy as jnp
from jax import lax
from jax.experimental import pallas as pl
from jax.experimental.pallas import tpu as pltpu
from jax.experimental.pallas import tpu_sc as plsc


def _build_kernel(B, S, V, E):
    info = plsc.get_sparse_core_info()
    NC, NS, L = info.num_cores, info.num_subcores, info.num_lanes
    NW = NC * NS
    assert S % NW == 0
    s_per_w = S // NW
    CH = min(8, s_per_w)
    assert s_per_w % CH == 0 and E % L == 0
    NCH = s_per_w // CH
    NGEN = 2 if NCH % 2 == 0 else 1

    mesh = plsc.VectorSubcoreMesh(core_axis_name="c", subcore_axis_name="s")

    scratch = [pltpu.VMEM((B, s_per_w), jnp.int32)]
    scratch += [pltpu.VMEM((B * CH, E), jnp.float32) for _ in range(NGEN)]
    scratch += [pltpu.VMEM((CH, E), jnp.float32) for _ in range(NGEN)]
    scratch += [pltpu.SemaphoreType.DMA for _ in range(3 * NGEN + 1)]

    @functools.partial(
        pl.kernel,
        mesh=mesh,
        out_type=jax.ShapeDtypeStruct((B, S, E), jnp.float32),
        scratch_types=scratch,
    )
    def emb_kernel(x_hbm, tok_hbm, pos_hbm, out_hbm, idx_v, *rest):
        bufs = list(rest[:NGEN])
        posb = list(rest[NGEN:2 * NGEN])
        sems = rest[2 * NGEN:]
        gsem = list(sems[:NGEN])
        osem = list(sems[NGEN:2 * NGEN])
        psem = list(sems[2 * NGEN:3 * NGEN])
        isem = sems[3 * NGEN]

        wid = lax.axis_index("s") * NC + lax.axis_index("c")
        s_base = wid * s_per_w

        def gather(c, g):
            return [
                pltpu.make_async_copy(
                    tok_hbm.at[idx_v.at[b, pl.ds(c * CH, CH)]],
                    bufs[g].at[pl.ds(b * CH, CH)],
                    gsem[g],
                )
                for b in range(B)
            ]

        def pos(c, g):
            return pltpu.make_async_copy(
                pos_hbm.at[pl.ds(s_base + c * CH, CH)],
                posb[g],
                psem[g],
            )

        def out(c, g):
            return [
                pltpu.make_async_copy(
                    bufs[g].at[pl.ds(b * CH, CH)],
                    out_hbm.at[b, pl.ds(s_base + c * CH, CH)],
                    osem[g],
                )
                for b in range(B)
            ]

        ihs = [
            pltpu.make_async_copy(
                x_hbm.at[b, pl.ds(s_base, s_per_w)], idx_v.at[b], isem
            )
            for b in range(B)
        ]
        for ih in ihs:
            ih.start()

        for g in range(NGEN):
            pos(g, g).start()
        for ih in ihs:
            ih.wait()
        for g in range(NGEN):
            for d in gather(g, g):
                d.start()

        @pl.loop(0, NCH, step=NGEN)
        def step_group(c0):
            for g in range(NGEN):
                c = c0 + g
                for d in gather(c, g):
                    d.wait()
                pos(c, g).wait()

                buf, pv_ref = bufs[g], posb[g]

                @plsc.parallel_loop(0, CH)
                def add_row(r, buf=buf, pv_ref=pv_ref):
                    for j in range(E // L):
                        sl = pl.ds(j * L, L)
                        pv = pv_ref[r, sl]
                        for b in range(B):
                            plsc.addupdate(buf.at[b * CH + r, sl], pv)

                for d in out(c, g):
                    d.start()

                @pl.when(c < NCH - NGEN)
                def _():
                    pos(c + NGEN, g).start()

                for d in out(c, g):
                    d.wait()

                @pl.when(c < NCH - NGEN)
                def _():
                    for d in gather(c + NGEN, g):
                        d.start()

    return emb_kernel


def kernel(x, tok_table, pos_table):
    B, S = x.shape
    V, E = tok_table.shape
    emb = _build_kernel(B, S, V, E)
    return emb(x.astype(jnp.int32), tok_table, pos_table)


# speedup vs baseline: 1.1049x; 1.0001x over previous
import functools

import jax
import jax.numpy as jnp
from jax import lax
from jax.experimental import pallas as pl
from jax.experimental.pallas import tpu as pltpu
from jax.experimental.pallas import tpu_sc as plsc


def _build_kernel(B, S, V, E):
    info = plsc.get_sparse_core_info()
    NC, NS, L = info.num_cores, info.num_subcores, info.num_lanes
    NW = NC * NS
    assert S % NW == 0
    s_per_w = S // NW
    CH = min(8, s_per_w)
    assert s_per_w % CH == 0 and E % L == 0
    NCH = s_per_w // CH
    NGEN = 2 if NCH % 2 == 0 else 1

    mesh = plsc.VectorSubcoreMesh(core_axis_name="c", subcore_axis_name="s")

    scratch = [pltpu.VMEM((B, s_per_w), jnp.int32)]
    scratch += [pltpu.VMEM((B * CH, E), jnp.float32) for _ in range(NGEN)]
    scratch += [pltpu.VMEM((CH, E), jnp.float32) for _ in range(NGEN)]
    scratch += [pltpu.SemaphoreType.DMA for _ in range(2 * NGEN + NGEN * B + 1)]

    @functools.partial(
        pl.kernel,
        mesh=mesh,
        out_type=jax.ShapeDtypeStruct((B, S, E), jnp.float32),
        scratch_types=scratch,
    )
    def emb_kernel(x_hbm, tok_hbm, pos_hbm, out_hbm, idx_v, *rest):
        bufs = list(rest[:NGEN])
        posb = list(rest[NGEN:2 * NGEN])
        sems = rest[2 * NGEN:]
        gsem = list(sems[:NGEN])
        oflat = list(sems[NGEN:NGEN + NGEN * B])
        osem = [oflat[g * B:(g + 1) * B] for g in range(NGEN)]
        psem = list(sems[NGEN + NGEN * B:2 * NGEN + NGEN * B])
        isem = sems[2 * NGEN + NGEN * B]

        wid = lax.axis_index("s") * NC + lax.axis_index("c")
        s_base = wid * s_per_w

        def gather(c, g):
            return [
                pltpu.make_async_copy(
                    tok_hbm.at[idx_v.at[b, pl.ds(c * CH, CH)]],
                    bufs[g].at[pl.ds(b * CH, CH)],
                    gsem[g],
                )
                for b in range(B)
            ]

        def pos(c, g):
            return pltpu.make_async_copy(
                pos_hbm.at[pl.ds(s_base + c * CH, CH)],
                posb[g],
                psem[g],
            )

        def out(c, g):
            return [
                pltpu.make_async_copy(
                    bufs[g].at[pl.ds(b * CH, CH)],
                    out_hbm.at[b, pl.ds(s_base + c * CH, CH)],
                    osem[g][b],
                )
                for b in range(B)
            ]

        ihs = [
            pltpu.make_async_copy(
                x_hbm.at[b, pl.ds(s_base, s_per_w)], idx_v.at[b], isem
            )
            for b in range(B)
        ]
        for ih in ihs:
            ih.start()

        for g in range(NGEN):
            pos(g, g).start()
        for ih in ihs:
            ih.wait()
        for g in range(NGEN):
            for d in gather(g, g):
                d.start()

        @pl.loop(0, NCH, step=NGEN)
        def step_group(c0):
            for g in range(NGEN):
                c = c0 + g
                for d in gather(c, g):
                    d.wait()
                pos(c, g).wait()

                buf, pv_ref = bufs[g], posb[g]

                @plsc.parallel_loop(0, CH)
                def add_row(r, buf=buf, pv_ref=pv_ref):
                    for j in range(E // L):
                        sl = pl.ds(j * L, L)
                        pv = pv_ref[r, sl]
                        for b in range(B):
                            plsc.addupdate(buf.at[b * CH + r, sl], pv)

                for d in out(c, g):
                    d.start()

                @pl.when(c < NCH - NGEN)
                def _():
                    pos(c + NGEN, g).start()

                ods = out(c, g)
                gds = gather(c + NGEN, g)
                for b in range(B):
                    ods[b].wait()

                    @pl.when(c < NCH - NGEN)
                    def _(b=b):
                        gds[b].start()

    return emb_kernel


def kernel(x, tok_table, pos_table):
    B, S = x.shape
    V, E = tok_table.shape
    emb = _build_kernel(B, S, V, E)
    return emb(x.astype(jnp.int32), tok_table, pos_table)
